# Initial kernel scaffold; baseline (speedup 1.0000x reference)
import jax
import jax.numpy as jnp
from jax.experimental import pallas as pl


def kernel(nullary, unary, binary, and_kernel, and_bias, or_nullary, or_unary, or_binary):
    raise NotImplementedError("write your pallas kernel here")


# speedup vs baseline: 1.6271x; 1.6271x over previous
import functools

import jax
import jax.numpy as jnp
from jax.experimental import pallas as pl

O = 12
V = 3
P = O * (O - 1) * (O - 2)
P0, U, BD = 16, 32, 16
R, C = 6, 64
RC = R * C
D = P0 + V * U + V * (V - 1) * BD

_HI = jax.lax.Precision.HIGHEST


def _dnf_kernel(nul_ref, una_ref, bin_ref, w_ref, bias_ref,
                orn_ref, oru_ref, orb_ref,
                conj_ref, outn_ref, outu_ref, outb_ref):
    w = w_ref[...]
    bias = bias_ref[...]

    base = jax.lax.dot_general(nul_ref[0], w[0:P0],
                               (((1,), (0,)), ((), ())),
                               precision=_HI) + bias
    una = una_ref[0]
    u_v = [jax.lax.dot_general(una, w[P0 + U * v: P0 + U * (v + 1)],
                               (((1,), (0,)), ((), ())),
                               precision=_HI) for v in range(V)]
    binb = bin_ref[0]
    boff = P0 + V * U
    bp = [jax.lax.dot_general(binb, w[boff + BD * s: boff + BD * (s + 1)],
                              (((1,), (0,)), ((), ())),
                              precision=_HI) for s in range(6)]

    def pair_row(fwd, rev, i):
        parts = []
        for j in range(O):
            if j == i:
                continue
            jj = j - (j > i)
            ii = i - (i > j)
            parts.append(fwd[i * (O - 1) + jj: i * (O - 1) + jj + 1]
                         + rev[j * (O - 1) + ii: j * (O - 1) + ii + 1])
        return jnp.concatenate(parts, axis=0)

    A_rows = [pair_row(bp[0], bp[2], i) for i in range(O)]
    B_rows = [pair_row(bp[1], bp[4], i) for i in range(O)]
    C_rows = [pair_row(bp[3], bp[5], i) for i in range(O)]

    null_maxes, u_rows, b_rows = [], [], []
    for o0 in range(O):
        others0 = [o for o in range(O) if o != o0]
        g_base = base + u_v[0][o0:o0 + 1]
        h = jnp.concatenate(
            [u_v[2][o2:o2 + 1] for o2 in others0], axis=0) + B_rows[o0]
        blocks = []
        for r1, o1 in enumerate(others0):
            def drop_row(arr, pos, n):
                parts = ([arr[:pos]] if pos > 0 else []) + \
                        ([arr[pos + 1:]] if pos < n - 1 else [])
                return parts[0] if len(parts) == 1 else \
                    jnp.concatenate(parts, axis=0)

            pos_o0 = o0 - (o0 > o1)
            c_sel = drop_row(C_rows[o1], pos_o0, O - 1)
            h_sel = drop_row(h, r1, O - 1)
            row = (g_base + u_v[1][o1:o1 + 1]
                   + A_rows[o0][r1:r1 + 1]) + h_sel + c_sel
            blocks.append(row)
        conj_o0 = jnp.tanh(jnp.concatenate(blocks, axis=0))
        conj_ref[0, o0 * 110:(o0 + 1) * 110, :] = conj_o0
        null_maxes.append(jnp.max(conj_o0[:, 0:2 * C], axis=0, keepdims=True))
        u_rows.append(jnp.max(conj_o0[:, 2 * C:4 * C], axis=0, keepdims=True))
        for r1 in range(O - 1):
            b_rows.append(jnp.max(
                conj_o0[r1 * 10:(r1 + 1) * 10, 4 * C:6 * C],
                axis=0, keepdims=True))

    null_max = functools.reduce(jnp.maximum, null_maxes)
    u_rules = jnp.concatenate(u_rows, axis=0)
    b_rules = jnp.concatenate(b_rows, axis=0)

    def disjoin(rules, or_ref):
        sig = jax.nn.sigmoid(or_ref[...])
        prod = rules * sig
        s0 = jnp.sum(prod[:, 0:C], axis=1, keepdims=True)
        s1 = jnp.sum(prod[:, C:2 * C], axis=1, keepdims=True)
        return jnp.tanh(jnp.concatenate([s0, s1], axis=1))

    outn_ref[0] = disjoin(null_max, orn_ref)
    outu_ref[0] = disjoin(u_rules, oru_ref)
    outb_ref[0] = disjoin(b_rules, orb_ref)


def kernel(nullary, unary, binary, and_kernel, and_bias,
           or_nullary, or_unary, or_binary):
    B = nullary.shape[0]
    w = and_kernel.transpose(2, 0, 1).reshape(D, RC)
    bias = and_bias.reshape(1, RC)
    bin2 = binary.reshape(B, O * (O - 1), BD)
    orn = or_nullary.reshape(1, 2 * C)
    oru = or_unary.reshape(1, 2 * C)
    orb = or_binary.reshape(1, 2 * C)
    nul3 = nullary.reshape(B, 1, P0)

    conj, outn, outu, outb = pl.pallas_call(
        _dnf_kernel,
        grid=(B,),
        in_specs=[
            pl.BlockSpec((1, 1, P0), lambda b: (b, 0, 0)),
            pl.BlockSpec((1, O, U), lambda b: (b, 0, 0)),
            pl.BlockSpec((1, O * (O - 1), BD), lambda b: (b, 0, 0)),
            pl.BlockSpec((D, RC), lambda b: (0, 0)),
            pl.BlockSpec((1, RC), lambda b: (0, 0)),
            pl.BlockSpec((1, 2 * C), lambda b: (0, 0)),
            pl.BlockSpec((1, 2 * C), lambda b: (0, 0)),
            pl.BlockSpec((1, 2 * C), lambda b: (0, 0)),
        ],
        out_specs=[
            pl.BlockSpec((1, P, RC), lambda b: (b, 0, 0)),
            pl.BlockSpec((1, 1, 2), lambda b: (b, 0, 0)),
            pl.BlockSpec((1, O, 2), lambda b: (b, 0, 0)),
            pl.BlockSpec((1, O * (O - 1), 2), lambda b: (b, 0, 0)),
        ],
        out_shape=[
            jax.ShapeDtypeStruct((B, P, RC), jnp.float32),
            jax.ShapeDtypeStruct((B, 1, 2), jnp.float32),
            jax.ShapeDtypeStruct((B, O, 2), jnp.float32),
            jax.ShapeDtypeStruct((B, O * (O - 1), 2), jnp.float32),
        ],
    )(nul3, unary, bin2, w, bias, orn, oru, orb)

    conjuncts = conj.reshape(B, P, R, C)
    out_binary = outb.reshape(B, O, O - 1, 2)
    return (outn.reshape(B, 2), outu, out_binary, conjuncts)


# speedup vs baseline: 1.9827x; 1.2186x over previous
import functools

import jax
import jax.numpy as jnp
from jax.experimental import pallas as pl
from jax.experimental.pallas import tpu as pltpu

O = 12
V = 3
P = O * (O - 1) * (O - 2)
P0, U, BD = 16, 32, 16
R, C = 6, 64
RC = R * C
D = P0 + V * U + V * (V - 1) * BD

_HI = jax.lax.Precision.DEFAULT


def _dnf_kernel(nul_ref, una_ref, bin_ref, w_ref, bias_ref,
                orn_ref, oru_ref, orb_ref,
                conj_ref, outn_ref, outu_ref, outb_ref):
    w = w_ref[...]
    bias = bias_ref[...]

    base = jax.lax.dot_general(nul_ref[0], w[0:P0],
                               (((1,), (0,)), ((), ())),
                               precision=_HI) + bias
    una = una_ref[0]
    wu = jnp.concatenate(
        [w[P0 + U * v: P0 + U * (v + 1)] for v in range(V)], axis=1)
    u_cat = jax.lax.dot_general(una, wu, (((1,), (0,)), ((), ())),
                                precision=_HI)
    u_v = [u_cat[:, RC * v: RC * (v + 1)] for v in range(V)]
    binb = bin_ref[0]
    boff = P0 + V * U
    wb = jnp.concatenate(
        [w[boff + BD * s: boff + BD * (s + 1)] for s in range(6)], axis=1)
    bp_cat = jax.lax.dot_general(binb, wb, (((1,), (0,)), ((), ())),
                                 precision=_HI)
    bp = [bp_cat[:, RC * s: RC * (s + 1)] for s in range(6)]

    def pair_row(fwd, rev, i):
        parts = []
        for j in range(O):
            if j == i:
                continue
            jj = j - (j > i)
            ii = i - (i > j)
            parts.append(fwd[i * (O - 1) + jj: i * (O - 1) + jj + 1]
                         + rev[j * (O - 1) + ii: j * (O - 1) + ii + 1])
        return jnp.concatenate(parts, axis=0)

    A_rows = [pair_row(bp[0], bp[2], i) for i in range(O)]
    B_rows = [pair_row(bp[1], bp[4], i) for i in range(O)]
    C_rows = [pair_row(bp[3], bp[5], i) for i in range(O)]

    null_maxes, u_rows, b_rows = [], [], []
    for o0 in range(O):
        others0 = [o for o in range(O) if o != o0]
        g_base = base + u_v[0][o0:o0 + 1]
        h = jnp.concatenate(
            [u_v[2][o2:o2 + 1] for o2 in others0], axis=0) + B_rows[o0]
        blocks = []
        for r1, o1 in enumerate(others0):
            def drop_row(arr, pos, n):
                parts = ([arr[:pos]] if pos > 0 else []) + \
                        ([arr[pos + 1:]] if pos < n - 1 else [])
                return parts[0] if len(parts) == 1 else \
                    jnp.concatenate(parts, axis=0)

            pos_o0 = o0 - (o0 > o1)
            c_sel = drop_row(C_rows[o1], pos_o0, O - 1)
            h_sel = drop_row(h, r1, O - 1)
            row = (g_base + u_v[1][o1:o1 + 1]
                   + A_rows[o0][r1:r1 + 1]) + h_sel + c_sel
            blocks.append(row)
        conj_o0 = jnp.tanh(jnp.concatenate(blocks, axis=0))
        conj_ref[0, o0 * 110:(o0 + 1) * 110, :] = conj_o0
        null_maxes.append(jnp.max(conj_o0[:, 0:2 * C], axis=0, keepdims=True))
        u_rows.append(jnp.max(conj_o0[:, 2 * C:4 * C], axis=0, keepdims=True))
        for r1 in range(O - 1):
            b_rows.append(jnp.max(
                conj_o0[r1 * 10:(r1 + 1) * 10, 4 * C:6 * C],
                axis=0, keepdims=True))

    null_max = functools.reduce(jnp.maximum, null_maxes)
    u_rules = jnp.concatenate(u_rows, axis=0)
    b_rules = jnp.concatenate(b_rows, axis=0)

    def disjoin(rules, or_ref):
        sig = jax.nn.sigmoid(or_ref[...])
        prod = rules * sig
        s0 = jnp.sum(prod[:, 0:C], axis=1, keepdims=True)
        s1 = jnp.sum(prod[:, C:2 * C], axis=1, keepdims=True)
        return jnp.tanh(jnp.concatenate([s0, s1], axis=1))

    outn_ref[0] = disjoin(null_max, orn_ref)
    outu_ref[0] = disjoin(u_rules, oru_ref)
    outb_ref[0] = disjoin(b_rules, orb_ref)


def kernel(nullary, unary, binary, and_kernel, and_bias,
           or_nullary, or_unary, or_binary):
    B = nullary.shape[0]
    w = and_kernel.transpose(2, 0, 1).reshape(D, RC)
    bias = and_bias.reshape(1, RC)
    bin2 = binary.reshape(B, O * (O - 1), BD)
    orn = or_nullary.reshape(1, 2 * C)
    oru = or_unary.reshape(1, 2 * C)
    orb = or_binary.reshape(1, 2 * C)
    nul3 = nullary.reshape(B, 1, P0)

    conj, outn, outu, outb = pl.pallas_call(
        _dnf_kernel,
        grid=(B,),
        in_specs=[
            pl.BlockSpec((1, 1, P0), lambda b: (b, 0, 0)),
            pl.BlockSpec((1, O, U), lambda b: (b, 0, 0)),
            pl.BlockSpec((1, O * (O - 1), BD), lambda b: (b, 0, 0)),
            pl.BlockSpec((D, RC), lambda b: (0, 0)),
            pl.BlockSpec((1, RC), lambda b: (0, 0)),
            pl.BlockSpec((1, 2 * C), lambda b: (0, 0)),
            pl.BlockSpec((1, 2 * C), lambda b: (0, 0)),
            pl.BlockSpec((1, 2 * C), lambda b: (0, 0)),
        ],
        out_specs=[
            pl.BlockSpec((1, P, RC), lambda b: (b, 0, 0)),
            pl.BlockSpec((1, 1, 2), lambda b: (b, 0, 0)),
            pl.BlockSpec((1, O, 2), lambda b: (b, 0, 0)),
            pl.BlockSpec((1, O * (O - 1), 2), lambda b: (b, 0, 0)),
        ],
        out_shape=[
            jax.ShapeDtypeStruct((B, P, RC), jnp.float32),
            jax.ShapeDtypeStruct((B, 1, 2), jnp.float32),
            jax.ShapeDtypeStruct((B, O, 2), jnp.float32),
            jax.ShapeDtypeStruct((B, O * (O - 1), 2), jnp.float32),
        ],
        compiler_params=pltpu.CompilerParams(
            dimension_semantics=("parallel",)),
    )(nul3, unary, bin2, w, bias, orn, oru, orb)

    conjuncts = conj.reshape(B, P, R, C)
    out_binary = outb.reshape(B, O, O - 1, 2)
    return (outn.reshape(B, 2), outu, out_binary, conjuncts)
